# baseline (speedup 1.0000x reference)
import functools

import jax
import jax.numpy as jnp
from jax import lax
from jax.experimental import pallas as pl
from jax.experimental.pallas import tpu as pltpu
from jax.experimental.pallas import tpu_sc as plsc

NUM_CORES = 2
NUM_SUBCORES = 16
NW = NUM_CORES * NUM_SUBCORES
NBUF = 2
PAIR = 2


@functools.partial(jax.jit, static_argnums=(2, 3, 4))
def _sc_gather(idx, table, bsz, f, d):
    per_w = bsz // NW
    npair = per_w // PAIR
    nout = npair // NBUF
    mesh = plsc.VectorSubcoreMesh(
        core_axis_name="c",
        subcore_axis_name="s",
        num_cores=NUM_CORES,
        num_subcores=NUM_SUBCORES,
    )

    @functools.partial(
        pl.kernel,
        out_type=jax.ShapeDtypeStruct((bsz, f, d), jnp.float32),
        mesh=mesh,
        scratch_types=[
            pltpu.VMEM((per_w, f), jnp.int32),
            pltpu.VMEM((NBUF, PAIR, f, d), jnp.float32),
        ]
        + [pltpu.SemaphoreType.DMA] * (2 * NBUF),
    )
    def k(idx_hbm, table_hbm, out_hbm, idx_v, rows_v, *sems):
        gsems = sems[:NBUF]
        osems = sems[NBUF:]
        wid = lax.axis_index("s") * NUM_CORES + lax.axis_index("c")
        base = wid * per_w

        pltpu.sync_copy(idx_hbm.at[pl.ds(base, per_w)], idx_v)

        def gather_start(p, b):
            for j in range(PAIR):
                pltpu.async_copy(
                    table_hbm.at[idx_v.at[p * PAIR + j]],
                    rows_v.at[b, j],
                    gsems[b],
                )

        def gather_wait(b):
            pltpu.make_async_copy(
                out_hbm.at[pl.ds(0, PAIR)],
                rows_v.at[b],
                gsems[b],
            ).wait()

        def store_start(p, b):
            pltpu.async_copy(
                rows_v.at[b], out_hbm.at[pl.ds(base + p * PAIR, PAIR)], osems[b]
            )

        def store_wait(b):
            pltpu.make_async_copy(
                rows_v.at[b], out_hbm.at[pl.ds(0, PAIR)], osems[b]
            ).wait()

        for b in range(NBUF):
            gather_start(b, b)

        def round_body(r, carry):
            for b in range(NBUF):
                gather_wait(b)
                store_start(r * NBUF + b, b)

            @pl.when(r < nout - 1)
            def _prefetch():
                for b in range(NBUF):
                    store_wait(b)
                    gather_start((r + 1) * NBUF + b, b)

            return carry

        lax.fori_loop(0, nout, round_body, 0)
        for b in range(NBUF):
            store_wait(b)

    return k(idx, table)


def kernel(indices, table):
    bsz, f = indices.shape
    d = table.shape[1]
    nchunk = 4
    cb = bsz // nchunk
    idx = indices.astype(jnp.int32)
    parts = [_sc_gather(idx[i * cb:(i + 1) * cb], table, cb, f, d)
             for i in range(nchunk)]
    return jnp.concatenate(parts, axis=0)


# speedup vs baseline: 1.6755x; 1.6755x over previous
import functools

import jax
import jax.numpy as jnp
from jax import lax
from jax.experimental import pallas as pl
from jax.experimental.pallas import tpu as pltpu
from jax.experimental.pallas import tpu_sc as plsc

NUM_CORES = 2
NUM_SUBCORES = 16
NW = NUM_CORES * NUM_SUBCORES
NBUF = 2
PAIR = 2


@functools.partial(jax.jit, static_argnums=(2, 3, 4))
def _sc_gather(idx, table, bsz, f, d):
    per_w = bsz // NW
    npair = per_w // PAIR
    nout = npair // NBUF
    mesh = plsc.VectorSubcoreMesh(
        core_axis_name="c",
        subcore_axis_name="s",
        num_cores=NUM_CORES,
        num_subcores=NUM_SUBCORES,
    )

    @functools.partial(
        pl.kernel,
        out_type=jax.ShapeDtypeStruct((bsz, f, d), jnp.float32),
        mesh=mesh,
        scratch_types=[
            pltpu.VMEM((per_w, f), jnp.int32),
            pltpu.VMEM((NBUF, PAIR, f, d), jnp.float32),
        ]
        + [pltpu.SemaphoreType.DMA] * (2 * NBUF),
    )
    def k(idx_hbm, table_hbm, out_hbm, idx_v, rows_v, *sems):
        gsems = sems[:NBUF]
        osems = sems[NBUF:]
        wid = lax.axis_index("s") * NUM_CORES + lax.axis_index("c")
        base = wid * per_w

        pltpu.sync_copy(idx_hbm.at[pl.ds(base, per_w)], idx_v)

        def gather_start(p, b):
            for j in range(PAIR):
                pltpu.async_copy(
                    table_hbm.at[idx_v.at[p * PAIR + j]],
                    rows_v.at[b, j],
                    gsems[b],
                )

        def gather_wait(b):
            pltpu.make_async_copy(
                out_hbm.at[pl.ds(0, PAIR)],
                rows_v.at[b],
                gsems[b],
            ).wait()

        def store_start(p, b):
            pltpu.async_copy(
                rows_v.at[b], out_hbm.at[pl.ds(base + p * PAIR, PAIR)], osems[b]
            )

        def store_wait(b):
            pltpu.make_async_copy(
                rows_v.at[b], out_hbm.at[pl.ds(0, PAIR)], osems[b]
            ).wait()

        for b in range(NBUF):
            gather_start(b, b)

        def round_body(r, carry):
            for b in range(NBUF):
                gather_wait(b)
                store_start(r * NBUF + b, b)

            @pl.when(r < nout - 1)
            def _prefetch():
                for b in range(NBUF):
                    store_wait(b)
                    gather_start((r + 1) * NBUF + b, b)

            return carry

        lax.fori_loop(0, nout, round_body, 0)
        for b in range(NBUF):
            store_wait(b)

    return k(idx, table)


def kernel(indices, table):
    bsz, f = indices.shape
    d = table.shape[1]
    return _sc_gather(indices.astype(jnp.int32), table, bsz, f, d)


# speedup vs baseline: 1.6913x; 1.0094x over previous
import functools

import jax
import jax.numpy as jnp
from jax import lax
from jax.experimental import pallas as pl
from jax.experimental.pallas import tpu as pltpu
from jax.experimental.pallas import tpu_sc as plsc

NUM_CORES = 2
NUM_SUBCORES = 16
NW = NUM_CORES * NUM_SUBCORES
NBUF = 2
PAIR = 4
NHALF = 4


@functools.partial(jax.jit, static_argnums=(2, 3, 4))
def _sc_gather(idx, table, bsz, f, d):
    per_w = bsz // NW
    per_h = per_w // NHALF
    npair = per_h // PAIR
    nout = npair // NBUF
    mesh = plsc.VectorSubcoreMesh(
        core_axis_name="c",
        subcore_axis_name="s",
        num_cores=NUM_CORES,
        num_subcores=NUM_SUBCORES,
    )

    @functools.partial(
        pl.kernel,
        out_type=jax.ShapeDtypeStruct((bsz, f, d), jnp.float32),
        mesh=mesh,
        scratch_types=[
            pltpu.VMEM((per_h, f), jnp.int32),
            pltpu.VMEM((NBUF, PAIR, f, d), jnp.float32),
        ]
        + [pltpu.SemaphoreType.DMA] * (2 * NBUF),
    )
    def k(idx_hbm, table_hbm, out_hbm, idx_v, rows_v, *sems):
        gsems = sems[:NBUF]
        osems = sems[NBUF:]
        wid = lax.axis_index("s") * NUM_CORES + lax.axis_index("c")

        def gather_wait(b):
            pltpu.make_async_copy(
                out_hbm.at[pl.ds(0, PAIR)],
                rows_v.at[b],
                gsems[b],
            ).wait()

        def store_wait(b):
            pltpu.make_async_copy(
                rows_v.at[b], out_hbm.at[pl.ds(0, PAIR)], osems[b]
            ).wait()

        for h in range(NHALF):
            base = wid * per_w + h * per_h

            pltpu.sync_copy(idx_hbm.at[pl.ds(base, per_h)], idx_v)

            def gather_start(p, b):
                for j in range(PAIR):
                    pltpu.async_copy(
                        table_hbm.at[idx_v.at[p * PAIR + j]],
                        rows_v.at[b, j],
                        gsems[b],
                    )

            def store_start(p, b):
                pltpu.async_copy(
                    rows_v.at[b],
                    out_hbm.at[pl.ds(base + p * PAIR, PAIR)],
                    osems[b],
                )

            for b in range(NBUF):
                gather_start(b, b)

            def round_body(r, carry):
                for b in range(NBUF):
                    gather_wait(b)
                    store_start(r * NBUF + b, b)

                @pl.when(r < nout - 1)
                def _prefetch():
                    for b in range(NBUF):
                        store_wait(b)
                        gather_start((r + 1) * NBUF + b, b)

                return carry

            lax.fori_loop(0, nout, round_body, 0)
            for b in range(NBUF):
                store_wait(b)

    return k(idx, table)


def kernel(indices, table):
    bsz, f = indices.shape
    d = table.shape[1]
    return _sc_gather(indices.astype(jnp.int32), table, bsz, f, d)
